# baseline (speedup 1.0000x reference)
import functools

import jax
import jax.numpy as jnp
from jax import lax
from jax.experimental import pallas as pl
from jax.experimental.pallas import tpu as pltpu
from jax.experimental.pallas import tpu_sc as plsc

B = 2
Q = 2048
N = 16384
K = 16
L = 16
NV = N // L
QPW = Q // 16
QB = 4
CHUNK = 4
SEED = 4
BUFW = 96

_INF = float("inf")


def _knn_body(qt_hbm, st_hbm, qtb_hbm, stb_hbm, out_hbm,
              xs, ys, zs, ssq, qx, qy, qz, qsq, buf, outv):
    c = lax.axis_index("c")
    s = lax.axis_index("s")
    qbase = s * QPW

    sb = c * (3 * N)
    pltpu.sync_copy(st_hbm.at[pl.ds(sb, N)], xs)
    pltpu.sync_copy(st_hbm.at[pl.ds(sb + N, N)], ys)
    pltpu.sync_copy(st_hbm.at[pl.ds(sb + 2 * N, N)], zs)
    qb = c * (3 * Q) + qbase
    pltpu.sync_copy(qt_hbm.at[pl.ds(qb, QPW)], qx)
    pltpu.sync_copy(qt_hbm.at[pl.ds(qb + Q, QPW)], qy)
    pltpu.sync_copy(qt_hbm.at[pl.ds(qb + 2 * Q, QPW)], qz)

    def ssq_body(i, carry):
        sl = pl.ds(i * L, L)
        a, b2, c2 = xs[sl], ys[sl], zs[sl]
        ssq[sl] = (a * a + b2 * b2) + c2 * c2
        return carry

    lax.fori_loop(0, NV, ssq_body, 0)

    def qsq_body(i, carry):
        sl = pl.ds(i * L, L)
        a, b2, c2 = qx[sl], qy[sl], qz[sl]
        qsq[sl] = (a * a + b2 * b2) + c2 * c2
        return carry

    lax.fori_loop(0, QPW // L, qsq_body, 0)

    pltpu.sync_copy(stb_hbm.at[pl.ds(sb, N)], xs)
    pltpu.sync_copy(stb_hbm.at[pl.ds(sb + N, N)], ys)
    pltpu.sync_copy(stb_hbm.at[pl.ds(sb + 2 * N, N)], zs)
    pltpu.sync_copy(qtb_hbm.at[pl.ds(qb, QPW)], qx)
    pltpu.sync_copy(qtb_hbm.at[pl.ds(qb + Q, QPW)], qy)
    pltpu.sync_copy(qtb_hbm.at[pl.ds(qb + 2 * Q, QPW)], qz)

    iota = lax.iota(jnp.int32, L)

    def run_group(qi0, qxs, qys, qzs, qss):
        def key_of(j, gx, gy, gz, gs):
            cross = (qxs[j] * gx + qys[j] * gy) + qzs[j] * gz
            return (qss[j] + gs) - 2.0 * cross

        def merge(j, idxv, tk, ti):
            gx = plsc.load_gather(xs, [idxv])
            gy = plsc.load_gather(ys, [idxv])
            gz = plsc.load_gather(zs, [idxv])
            gs = plsc.load_gather(ssq, [idxv])
            key = key_of(j, gx, gy, gz, gs)
            return _bitonic_merge(key, idxv, tk, ti)

        def drain(j, cnt, tau, tk, ti):
            def w_body(st):
                rp, _, tk_, ti_ = st
                idxv = buf[pl.ds(j * BUFW + rp, L)]
                tk_, ti_ = merge(j, idxv, tk_, ti_)
                return rp + L, jnp.max(tk_), tk_, ti_

            rp, tau, tk, ti = lax.while_loop(
                lambda st: cnt - st[0] >= K, w_body, (0, tau, tk, ti))

            @pl.when(rp > 0)
            def _():
                buf[pl.ds(j * BUFW, L)] = buf[pl.ds(j * BUFW + rp, L)]

            return cnt - rp, tau, tk, ti

        def chunk_body(ci, st):
            st = list(st)
            base = ci * (CHUNK * L)
            taus = [st[j][1] for j in range(QB)]

            zero = jnp.zeros((L,), jnp.int32)

            @plsc.parallel_loop(0, CHUNK,
                                carry=tuple(zero + s[0] for s in st),
                                unroll=CHUNK)
            def cnts(v, cnts_in):
                off = base + v * L
                sl = pl.ds(off, L)
                sx, sy, sz, ss = xs[sl], ys[sl], zs[sl], ssq[sl]
                iv = iota + off
                out = []
                for j in range(QB):
                    d2 = key_of(j, sx, sy, sz, ss)
                    m = d2 < taus[j]
                    inc = plsc.cumsum(jnp.where(m, 1, 0).astype(jnp.int32))
                    pos = cnts_in[j] + (inc + (j * BUFW - 1))
                    plsc.store_scatter(buf, [pos], iv, mask=m)
                    out.append(cnts_in[j]
                               + plsc.all_reduce_population_count(m))
                return tuple(out)

            for j in range(QB):
                st[j] = drain(j, cnts[j][0], *st[j][1:])
            return tuple(st)

        init = tuple((jnp.int32(0), jnp.float32(_INF),
                      jnp.full((L,), _INF, jnp.float32),
                      jnp.zeros((L,), jnp.int32)) for _ in range(QB))
        st = lax.fori_loop(0, NV // CHUNK, chunk_body, init)

        for j in range(QB):
            cnt, tau, tk, ti = st[j]
            valid = iota < cnt
            idxv = jnp.where(valid, buf[pl.ds(j * BUFW, L)], 0)
            gx = plsc.load_gather(xs, [idxv])
            gy = plsc.load_gather(ys, [idxv])
            gz = plsc.load_gather(zs, [idxv])
            gs = plsc.load_gather(ssq, [idxv])
            key = jnp.where(valid, key_of(j, gx, gy, gz, gs), _INF)
            tk, ti = _bitonic_merge(key, idxv, tk, ti)
            outv[pl.ds((qi0 + j) * K, K)] = ti

    def sgroup_body(sg, carry):
        qb16 = sg * L
        qxv = qx[pl.ds(qb16, L)]
        qyv = qy[pl.ds(qb16, L)]
        qzv = qz[pl.ds(qb16, L)]
        qsv = qsq[pl.ds(qb16, L)]
        for sub in range(L // QB):
            lanes = [sub * QB + j for j in range(QB)]
            run_group(qb16 + sub * QB,
                      [qxv[l] for l in lanes], [qyv[l] for l in lanes],
                      [qzv[l] for l in lanes], [qsv[l] for l in lanes])
        return carry

    lax.fori_loop(0, QPW // L, sgroup_body, 0)
    pltpu.sync_copy(outv, out_hbm.at[pl.ds(c * (Q * K) + qbase * K, QPW * K)])


def _bitonic_merge(key, idxv, tk, ti):
    sk, sv = plsc.sort_key_val(key, idxv)
    rk = lax.rev(sk, (0,))
    rv = lax.rev(sv, (0,))
    sel = tk <= rk
    mk = jnp.where(sel, tk, rk)
    mv = jnp.where(sel, ti, rv)
    return plsc.sort_key_val(mk, mv)


@jax.jit
def _knn(qt, st, qtb, stb):
    mesh = plsc.VectorSubcoreMesh(core_axis_name="c", subcore_axis_name="s",
                                  num_cores=2, num_subcores=16)
    f = pl.kernel(
        _knn_body,
        out_type=jax.ShapeDtypeStruct((B * Q * K,), jnp.int32),
        mesh=mesh,
        compiler_params=pltpu.CompilerParams(needs_layout_passes=False),
        scratch_types=[
            pltpu.VMEM((N,), jnp.float32),
            pltpu.VMEM((N,), jnp.float32),
            pltpu.VMEM((N,), jnp.float32),
            pltpu.VMEM((N,), jnp.float32),
            pltpu.VMEM((QPW,), jnp.float32),
            pltpu.VMEM((QPW,), jnp.float32),
            pltpu.VMEM((QPW,), jnp.float32),
            pltpu.VMEM((QPW,), jnp.float32),
            pltpu.VMEM((QB * BUFW,), jnp.int32),
            pltpu.VMEM((QPW * K,), jnp.int32),
        ],
    )
    return f(qt, st, qtb, stb)


def _bf16_round(x):
    u = lax.bitcast_convert_type(x, jnp.uint32)
    r = (u + jnp.uint32(0x7FFF) + ((u >> 16) & jnp.uint32(1)))
    r = r & jnp.uint32(0xFFFF0000)
    return lax.bitcast_convert_type(r, jnp.float32)


def kernel(q_points, support):
    qt = q_points.transpose(0, 2, 1).reshape(-1)
    st = support.transpose(0, 2, 1).reshape(-1)
    qtb = _bf16_round(qt)
    stb = _bf16_round(st)
    return _knn(qt, st, qtb, stb).reshape(B, Q, K)


# speedup vs baseline: 1.3952x; 1.3952x over previous
import functools

import jax
import jax.numpy as jnp
from jax import lax
from jax.experimental import pallas as pl
from jax.experimental.pallas import tpu as pltpu
from jax.experimental.pallas import tpu_sc as plsc

B = 2
Q = 2048
N = 16384
K = 16
L = 16
NV = N // L
QPW = Q // 16
QB = 4
CHUNK = 4
SEED = 4
BUFW = 96

_INF = float("inf")


def _knn_body(qt_hbm, st_hbm, qtb_hbm, stb_hbm, out_hbm,
              xs, ys, zs, ssq, qx, qy, qz, qsq, buf, outv):
    c = lax.axis_index("c")
    s = lax.axis_index("s")
    qbase = s * QPW

    sb = c * (3 * N)
    pltpu.sync_copy(st_hbm.at[pl.ds(sb, N)], xs)
    pltpu.sync_copy(st_hbm.at[pl.ds(sb + N, N)], ys)
    pltpu.sync_copy(st_hbm.at[pl.ds(sb + 2 * N, N)], zs)
    qb = c * (3 * Q) + qbase
    pltpu.sync_copy(qt_hbm.at[pl.ds(qb, QPW)], qx)
    pltpu.sync_copy(qt_hbm.at[pl.ds(qb + Q, QPW)], qy)
    pltpu.sync_copy(qt_hbm.at[pl.ds(qb + 2 * Q, QPW)], qz)

    def ssq_body(i, carry):
        sl = pl.ds(i * L, L)
        a, b2, c2 = xs[sl], ys[sl], zs[sl]
        ssq[sl] = (a * a + b2 * b2) + c2 * c2
        return carry

    lax.fori_loop(0, NV, ssq_body, 0)

    def qsq_body(i, carry):
        sl = pl.ds(i * L, L)
        a, b2, c2 = qx[sl], qy[sl], qz[sl]
        qsq[sl] = (a * a + b2 * b2) + c2 * c2
        return carry

    lax.fori_loop(0, QPW // L, qsq_body, 0)

    pltpu.sync_copy(stb_hbm.at[pl.ds(sb, N)], xs)
    pltpu.sync_copy(stb_hbm.at[pl.ds(sb + N, N)], ys)
    pltpu.sync_copy(stb_hbm.at[pl.ds(sb + 2 * N, N)], zs)
    pltpu.sync_copy(qtb_hbm.at[pl.ds(qb, QPW)], qx)
    pltpu.sync_copy(qtb_hbm.at[pl.ds(qb + Q, QPW)], qy)
    pltpu.sync_copy(qtb_hbm.at[pl.ds(qb + 2 * Q, QPW)], qz)

    iota = lax.iota(jnp.int32, L)

    def run_group(qi0, qxs, qys, qzs, qss):
        def key_of(j, gx, gy, gz, gs):
            cross = (qxs[j] * gx + qys[j] * gy) + qzs[j] * gz
            return (qss[j] + gs) - 2.0 * cross

        def filt_of(j, gx, gy, gz, gs):
            cross = (qxs[j] * gx + qys[j] * gy) + qzs[j] * gz
            return gs - 2.0 * cross

        def shift_tau(j, tk):
            return jnp.max((tk - qss[j]) + jnp.float32(1e-4))

        def merge(j, idxv, tk, ti):
            gx = plsc.load_gather(xs, [idxv])
            gy = plsc.load_gather(ys, [idxv])
            gz = plsc.load_gather(zs, [idxv])
            gs = plsc.load_gather(ssq, [idxv])
            key = key_of(j, gx, gy, gz, gs)
            return _bitonic_merge(key, idxv, tk, ti)

        def drain(j, cnt, tau, tk, ti):
            def w_body(st):
                rp, _, tk_, ti_ = st
                idxv = buf[pl.ds(j * BUFW + rp, L)]
                tk_, ti_ = merge(j, idxv, tk_, ti_)
                return rp + L, shift_tau(j, tk_), tk_, ti_

            rp, tau, tk, ti = lax.while_loop(
                lambda st: cnt - st[0] >= K, w_body, (0, tau, tk, ti))

            @pl.when(rp > 0)
            def _():
                buf[pl.ds(j * BUFW, L)] = buf[pl.ds(j * BUFW + rp, L)]

            return cnt - rp, tau, tk, ti

        def chunk_body(ci, st):
            st = list(st)
            base = ci * (CHUNK * L)
            taus = [st[j][1] for j in range(QB)]

            @plsc.parallel_loop(0, CHUNK, carry=tuple(s[0] for s in st),
                                unroll=CHUNK)
            def cnts(v, cnts_in):
                off = base + v * L
                sl = pl.ds(off, L)
                sx, sy, sz, ss = xs[sl], ys[sl], zs[sl], ssq[sl]
                iv = iota + off
                out = []
                for j in range(QB):
                    f = filt_of(j, sx, sy, sz, ss)
                    m = f < taus[j]
                    inc = plsc.cumsum(jnp.where(m, 1, 0).astype(jnp.int32))
                    pos = (j * BUFW - 1 + cnts_in[j]) + inc
                    plsc.store_scatter(buf, [pos], iv, mask=m)
                    out.append(cnts_in[j] + inc[L - 1])
                return tuple(out)

            for j in range(QB):
                st[j] = drain(j, cnts[j], *st[j][1:])
            return tuple(st)

        init = tuple((jnp.int32(0), jnp.float32(_INF),
                      jnp.full((L,), _INF, jnp.float32),
                      jnp.zeros((L,), jnp.int32)) for _ in range(QB))
        st = lax.fori_loop(0, NV // CHUNK, chunk_body, init)

        for j in range(QB):
            cnt, tau, tk, ti = st[j]
            valid = iota < cnt
            idxv = jnp.where(valid, buf[pl.ds(j * BUFW, L)], 0)
            gx = plsc.load_gather(xs, [idxv])
            gy = plsc.load_gather(ys, [idxv])
            gz = plsc.load_gather(zs, [idxv])
            gs = plsc.load_gather(ssq, [idxv])
            key = jnp.where(valid, key_of(j, gx, gy, gz, gs), _INF)
            tk, ti = _bitonic_merge(key, idxv, tk, ti)
            outv[pl.ds((qi0 + j) * K, K)] = ti

    def sgroup_body(sg, carry):
        qb16 = sg * L
        qxv = qx[pl.ds(qb16, L)]
        qyv = qy[pl.ds(qb16, L)]
        qzv = qz[pl.ds(qb16, L)]
        qsv = qsq[pl.ds(qb16, L)]
        for sub in range(L // QB):
            lanes = [sub * QB + j for j in range(QB)]
            run_group(qb16 + sub * QB,
                      [qxv[l] for l in lanes], [qyv[l] for l in lanes],
                      [qzv[l] for l in lanes], [qsv[l] for l in lanes])
        return carry

    lax.fori_loop(0, QPW // L, sgroup_body, 0)
    pltpu.sync_copy(outv, out_hbm.at[pl.ds(c * (Q * K) + qbase * K, QPW * K)])


def _bitonic_merge(key, idxv, tk, ti):
    sk, sv = plsc.sort_key_val(key, idxv)
    rk = lax.rev(sk, (0,))
    rv = lax.rev(sv, (0,))
    sel = tk <= rk
    mk = jnp.where(sel, tk, rk)
    mv = jnp.where(sel, ti, rv)
    return plsc.sort_key_val(mk, mv)


@jax.jit
def _knn(qt, st, qtb, stb):
    mesh = plsc.VectorSubcoreMesh(core_axis_name="c", subcore_axis_name="s",
                                  num_cores=2, num_subcores=16)
    f = pl.kernel(
        _knn_body,
        out_type=jax.ShapeDtypeStruct((B * Q * K,), jnp.int32),
        mesh=mesh,
        compiler_params=pltpu.CompilerParams(needs_layout_passes=False),
        scratch_types=[
            pltpu.VMEM((N,), jnp.float32),
            pltpu.VMEM((N,), jnp.float32),
            pltpu.VMEM((N,), jnp.float32),
            pltpu.VMEM((N,), jnp.float32),
            pltpu.VMEM((QPW,), jnp.float32),
            pltpu.VMEM((QPW,), jnp.float32),
            pltpu.VMEM((QPW,), jnp.float32),
            pltpu.VMEM((QPW,), jnp.float32),
            pltpu.VMEM((QB * BUFW,), jnp.int32),
            pltpu.VMEM((QPW * K,), jnp.int32),
        ],
    )
    return f(qt, st, qtb, stb)


def _bf16_round(x):
    u = lax.bitcast_convert_type(x, jnp.uint32)
    r = (u + jnp.uint32(0x7FFF) + ((u >> 16) & jnp.uint32(1)))
    r = r & jnp.uint32(0xFFFF0000)
    return lax.bitcast_convert_type(r, jnp.float32)


def kernel(q_points, support):
    qt = q_points.transpose(0, 2, 1).reshape(-1)
    st = support.transpose(0, 2, 1).reshape(-1)
    qtb = _bf16_round(qt)
    stb = _bf16_round(st)
    return _knn(qt, st, qtb, stb).reshape(B, Q, K)
